# baseline (speedup 1.0000x reference)
import jax
import jax.numpy as jnp
from jax import lax
from jax.experimental import pallas as pl
from jax.experimental.pallas import tpu as pltpu
from jax.experimental.pallas import tpu_sc as plsc
import functools

_B = 4
_S = 2048
_HID = 768
_NC = 2
_NS = 16
_NW = _NC * _NS
_SPW = _S // _NW
_C = 32
_HPB = _SPW // _C
_NCHUNK = _B * _HPB
_NBUF = 3


def _make_kernel():
    mesh = plsc.VectorSubcoreMesh(core_axis_name="c", subcore_axis_name="s")

    @functools.partial(
        pl.kernel,
        mesh=mesh,
        out_type=jax.ShapeDtypeStruct((_B, _S, _HID), jnp.float32),
        scratch_types=(
            [pltpu.VMEM((_B, _SPW), jnp.int32),
             pltpu.VMEM((_SPW, _HID), jnp.float32)]
            + [pltpu.VMEM((_C, _HID), jnp.float32) for _ in range(_NBUF)]
            + [pltpu.SemaphoreType.DMA for _ in range(2 * _NBUF + 1)]
        ),
    )
    def emb_kernel(ids_hbm, word_hbm, pos_hbm, out_hbm, idx_v, pos_v, *rest):
        bufs = rest[:_NBUF]
        gsems = rest[_NBUF:2 * _NBUF]
        ssems = rest[2 * _NBUF:3 * _NBUF]
        psem = rest[3 * _NBUF]

        wid = lax.axis_index("s") * _NC + lax.axis_index("c")
        s0 = wid * _SPW

        pos_desc = pltpu.async_copy(pos_hbm.at[pl.ds(s0, _SPW)], pos_v, psem)
        for b in range(_B):
            pltpu.sync_copy(ids_hbm.at[b, pl.ds(s0, _SPW)], idx_v.at[b])

        gdescs = [None] * _NCHUNK
        sdescs = [None] * _NCHUNK

        def start_gather(c):
            b, h = c // _HPB, c % _HPB
            idx_ref = idx_v.at[b, pl.ds(h * _C, _C)]
            gdescs[c] = pltpu.async_copy(
                word_hbm.at[idx_ref], bufs[c % _NBUF], gsems[c % _NBUF])

        start_gather(0)
        pos_desc.wait()

        for c in range(_NCHUNK):
            if c + 1 < _NCHUNK:
                if c - 2 >= 0:
                    sdescs[c - 2].wait()
                start_gather(c + 1)
            b, h = c // _HPB, c % _HPB
            gdescs[c].wait()
            buf = bufs[c % _NBUF]

            @plsc.parallel_loop(0, _C, unroll=2)
            def add_row(i):
                for j in range(_HID // 16):
                    sl = pl.ds(j * 16, 16)
                    plsc.addupdate(buf.at[i, sl], pos_v[h * _C + i, sl])

            sdescs[c] = pltpu.async_copy(
                buf, out_hbm.at[b, pl.ds(s0 + h * _C, _C)], ssems[c % _NBUF])

        for c in range(_NCHUNK - 3, _NCHUNK):
            sdescs[c].wait()

    return emb_kernel


_emb_kernel = _make_kernel()


@jax.jit
def kernel(input_ids, word_embeddings, position_embeddings):
    ids = input_ids.astype(jnp.int32)
    return _emb_kernel(ids, word_embeddings, position_embeddings)


# speedup vs baseline: 1.0530x; 1.0530x over previous
import jax
import jax.numpy as jnp
from jax import lax
from jax.experimental import pallas as pl
from jax.experimental.pallas import tpu as pltpu
from jax.experimental.pallas import tpu_sc as plsc
import functools

_B = 4
_S = 2048
_HID = 768
_NC = 2
_NS = 16
_NW = _NC * _NS
_SPW = _S // _NW
_C = 32
_HPB = _SPW // _C
_NCHUNK = _B * _HPB
_NBUF = 3


def _make_kernel():
    mesh = plsc.VectorSubcoreMesh(core_axis_name="c", subcore_axis_name="s")

    @functools.partial(
        pl.kernel,
        mesh=mesh,
        out_type=jax.ShapeDtypeStruct((_B, _S, _HID), jnp.float32),
        scratch_types=(
            [pltpu.VMEM((_B, _SPW), jnp.int32),
             pltpu.VMEM((_SPW, _HID), jnp.float32)]
            + [pltpu.VMEM((_C, _HID), jnp.float32) for _ in range(_NBUF)]
            + [pltpu.SemaphoreType.DMA for _ in range(2 * _NBUF + 1)]
        ),
    )
    def emb_kernel(ids_hbm, word_hbm, pos_hbm, out_hbm, idx_v, pos_v, *rest):
        bufs = rest[:_NBUF]
        gsems = rest[_NBUF:2 * _NBUF]
        ssems = rest[2 * _NBUF:3 * _NBUF]
        psem = rest[3 * _NBUF]

        wid = lax.axis_index("s") * _NC + lax.axis_index("c")
        s0 = wid * _SPW

        pos_desc = pltpu.async_copy(pos_hbm.at[pl.ds(s0, _SPW)], pos_v, psem)
        for b in range(_B):
            pltpu.sync_copy(ids_hbm.at[b, pl.ds(s0, _SPW)], idx_v.at[b])

        gdescs = [None] * _NCHUNK
        sdescs = [None] * _NCHUNK

        def start_gather(c):
            b, h = c // _HPB, c % _HPB
            idx_ref = idx_v.at[b, pl.ds(h * _C, _C)]
            gdescs[c] = pltpu.async_copy(
                word_hbm.at[idx_ref], bufs[c % _NBUF], gsems[c % _NBUF])

        start_gather(0)
        pos_desc.wait()

        for c in range(_NCHUNK):
            if c + 1 < _NCHUNK:
                if c - 2 >= 0:
                    sdescs[c - 2].wait()
                start_gather(c + 1)
            b, h = c // _HPB, c % _HPB
            gdescs[c].wait()
            buf = bufs[c % _NBUF]

            @plsc.parallel_loop(0, _C)
            def add_row(i):
                for j in range(_HID // 16):
                    sl = pl.ds(j * 16, 16)
                    plsc.addupdate(buf.at[i, sl], pos_v[h * _C + i, sl])

            sdescs[c] = pltpu.async_copy(
                buf, out_hbm.at[b, pl.ds(s0 + h * _C, _C)], ssems[c % _NBUF])

        for c in range(_NCHUNK - 3, _NCHUNK):
            sdescs[c].wait()

    return emb_kernel


_emb_kernel = _make_kernel()


@jax.jit
def kernel(input_ids, word_embeddings, position_embeddings):
    ids = input_ids.astype(jnp.int32)
    return _emb_kernel(ids, word_embeddings, position_embeddings)


# speedup vs baseline: 1.1020x; 1.0465x over previous
import jax
import jax.numpy as jnp
from jax import lax
from jax.experimental import pallas as pl
from jax.experimental.pallas import tpu as pltpu
from jax.experimental.pallas import tpu_sc as plsc
import functools

_B = 4
_S = 2048
_HID = 768
_NC = 2
_NS = 16
_NW = _NC * _NS
_SPW = _S // _NW
_C = 32
_HPB = _SPW // _C
_NCHUNK = _B * _HPB
_NBUF = 3


def _make_kernel():
    mesh = plsc.VectorSubcoreMesh(core_axis_name="c", subcore_axis_name="s")

    @functools.partial(
        pl.kernel,
        mesh=mesh,
        out_type=jax.ShapeDtypeStruct((_B, _S, _HID), jnp.float32),
        scratch_types=(
            [pltpu.VMEM((_B, _SPW), jnp.int32),
             pltpu.VMEM((_SPW, _HID), jnp.float32)]
            + [pltpu.VMEM((_C, _HID), jnp.float32) for _ in range(_NBUF)]
            + [pltpu.SemaphoreType.DMA for _ in range(2 * _NBUF + 1)]
        ),
    )
    def emb_kernel(ids_hbm, word_hbm, pos_hbm, out_hbm, idx_v, pos_v, *rest):
        bufs = rest[:_NBUF]
        gsems = rest[_NBUF:2 * _NBUF]
        ssems = rest[2 * _NBUF:3 * _NBUF]
        psem = rest[3 * _NBUF]

        wid = lax.axis_index("s") * _NC + lax.axis_index("c")
        s0 = wid * _SPW

        pos_desc = pltpu.async_copy(pos_hbm.at[pl.ds(s0, _SPW)], pos_v, psem)
        for b in range(_B):
            pltpu.sync_copy(ids_hbm.at[b, pl.ds(s0, _SPW)], idx_v.at[b])

        gdescs = [None] * _NCHUNK
        sdescs = [None] * _NCHUNK

        def start_gather(c):
            b, h = c // _HPB, c % _HPB
            idx_ref = idx_v.at[b, pl.ds(h * _C, _C)]
            gdescs[c] = pltpu.async_copy(
                word_hbm.at[idx_ref], bufs[c % _NBUF], gsems[c % _NBUF])

        start_gather(0)
        pos_desc.wait()

        for c in range(_NCHUNK):
            if c + 1 < _NCHUNK:
                if c - 2 >= 0:
                    sdescs[c - 2].wait()
                start_gather(c + 1)
            b, h = c // _HPB, c % _HPB
            gdescs[c].wait()
            buf = bufs[c % _NBUF]

            @plsc.parallel_loop(0, _HID // 16)
            def add_col(j):
                sl = pl.ds(j * 16, 16)
                for i in range(_C):
                    plsc.addupdate(buf.at[i, sl], pos_v[h * _C + i, sl])

            sdescs[c] = pltpu.async_copy(
                buf, out_hbm.at[b, pl.ds(s0 + h * _C, _C)], ssems[c % _NBUF])

        for c in range(_NCHUNK - 3, _NCHUNK):
            sdescs[c].wait()

    return emb_kernel


_emb_kernel = _make_kernel()


@jax.jit
def kernel(input_ids, word_embeddings, position_embeddings):
    ids = input_ids.astype(jnp.int32)
    return _emb_kernel(ids, word_embeddings, position_embeddings)
